# baseline (speedup 1.0000x reference)
import functools

import jax
import jax.numpy as jnp
from jax import lax
from jax.experimental import pallas as pl
from jax.experimental.pallas import tpu as pltpu
from jax.experimental.pallas import tpu_sc as plsc

_NUM_CORES = 2
_NUM_SUBCORES = 16
_NUM_WORKERS = _NUM_CORES * _NUM_SUBCORES


@functools.partial(jax.jit, static_argnums=(2, 3))
def _gather_rows(table, idx, n, d):
    b_per_w = n // _NUM_WORKERS
    mesh = plsc.VectorSubcoreMesh(core_axis_name="c", subcore_axis_name="s")

    @functools.partial(
        pl.kernel,
        mesh=mesh,
        out_type=jax.ShapeDtypeStruct((n, d), jnp.float32),
        scratch_types=[
            pltpu.VMEM((b_per_w,), jnp.int32),
            pltpu.VMEM((b_per_w, d), jnp.float32),
            pltpu.SemaphoreType.DMA,
        ],
        compiler_params=pltpu.CompilerParams(use_tc_tiling_on_sc=False),
    )
    def gather(table_hbm, idx_hbm, out_hbm, idx_v, rows_v, sem):
        wid = lax.axis_index("s") * _NUM_CORES + lax.axis_index("c")
        base = wid * b_per_w
        pltpu.sync_copy(idx_hbm.at[pl.ds(base, b_per_w)], idx_v)
        pltpu.async_copy(table_hbm.at[idx_v], rows_v, sem).wait()
        pltpu.sync_copy(rows_v, out_hbm.at[pl.ds(base, b_per_w)])

    return gather(table, idx)


def kernel(inputs, embedding):
    b, l = inputs.shape
    v, d = embedding.shape
    n = b * l
    idx = inputs.reshape(-1).astype(jnp.int32)
    rows = _gather_rows(embedding, idx, n, d)
    return rows.reshape(b, l * d)


# speedup vs baseline: 1.8972x; 1.8972x over previous
import functools

import jax
import jax.numpy as jnp
from jax import lax
from jax.experimental import pallas as pl
from jax.experimental.pallas import tpu as pltpu
from jax.experimental.pallas import tpu_sc as plsc

_NUM_CORES = 2
_NUM_SUBCORES = 16
_NUM_WORKERS = _NUM_CORES * _NUM_SUBCORES
_P1_CHUNK = 98304


@functools.partial(jax.jit, static_argnums=(2,))
def _gather_t(table_t, idx_t, v):
    d, vv = table_t.shape
    l_fields, b = idx_t.shape
    m_rows = l_fields * d
    mesh = plsc.VectorSubcoreMesh(core_axis_name="c", subcore_axis_name="s")

    @functools.partial(
        pl.kernel,
        mesh=mesh,
        out_type=[
            jax.ShapeDtypeStruct((m_rows, b), jnp.float32),
            jax.ShapeDtypeStruct((d * v,), jnp.float32),
        ],
        scratch_types=[
            pltpu.VMEM((b,), jnp.int32),
            pltpu.VMEM((b,), jnp.float32),
            pltpu.VMEM((_P1_CHUNK,), jnp.float32),
            pltpu.VMEM((1, max(v % 128, 1)), jnp.float32),
            pltpu.SemaphoreType.DMA,
        ],
        compiler_params=pltpu.CompilerParams(needs_layout_passes=False),
    )
    def gather(table_hbm, idx_hbm, out_hbm, scr_hbm, idx_v, vals_v, chunk_v,
               tail_v, sem):
        w = lax.axis_index("s") * _NUM_CORES + lax.axis_index("c")
        row = scr_hbm.at[pl.ds(w * v, v)]

        def do_stage(t, _):
            off = t * _P1_CHUNK
            pltpu.sync_copy(table_hbm.at[w].at[pl.ds(off, _P1_CHUNK)], chunk_v)
            pltpu.sync_copy(chunk_v, row.at[pl.ds(off, _P1_CHUNK)])
            return 0

        n_full = v // _P1_CHUNK
        lax.fori_loop(0, n_full, do_stage, 0)
        rem_off = n_full * _P1_CHUNK
        rem_aligned = ((v - rem_off) // 128) * 128
        if rem_aligned:
            src = table_hbm.at[w].at[pl.ds(rem_off, rem_aligned)]
            dst_v = chunk_v.at[pl.ds(0, rem_aligned)]
            pltpu.sync_copy(src, dst_v)
            pltpu.sync_copy(dst_v, row.at[pl.ds(rem_off, rem_aligned)])
        tail_off = rem_off + rem_aligned
        tail = v - tail_off
        if tail:
            src = table_hbm.at[pl.ds(w, 1), pl.ds(tail_off, tail)]
            pltpu.sync_copy(src, tail_v)
            pltpu.sync_copy(
                tail_v.at[0], row.at[pl.ds(tail_off, tail)]
            )

        def do_field(r, _):
            pltpu.sync_copy(idx_hbm.at[r], idx_v)
            pltpu.async_copy(row.at[idx_v], vals_v, sem).wait()
            pltpu.sync_copy(vals_v, out_hbm.at[r * d + w])
            return 0

        lax.fori_loop(0, l_fields, do_field, 0)

    return gather(table_t, idx_t)[0]


def kernel(inputs, embedding):
    b, l = inputs.shape
    v, d = embedding.shape
    idx_t = inputs.T.astype(jnp.int32)
    table_t = embedding.T
    out_t = _gather_t(table_t, idx_t, v)
    return out_t.T


# speedup vs baseline: 2.0344x; 1.0723x over previous
import functools

import jax
import jax.numpy as jnp
from jax import lax
from jax.experimental import pallas as pl
from jax.experimental.pallas import tpu as pltpu
from jax.experimental.pallas import tpu_sc as plsc

_NUM_CORES = 2
_NUM_SUBCORES = 16
_NUM_WORKERS = _NUM_CORES * _NUM_SUBCORES
_P1_CHUNK = 57216


@functools.partial(jax.jit, static_argnums=(2,))
def _gather_t(table_t, idx_t, v):
    d, _ = table_t.shape
    l_fields, b = idx_t.shape
    m_rows = l_fields * d
    mesh = plsc.VectorSubcoreMesh(core_axis_name="c", subcore_axis_name="s")

    chunks = [(t * _P1_CHUNK, _P1_CHUNK) for t in range(v // _P1_CHUNK)]
    rem_off = (v // _P1_CHUNK) * _P1_CHUNK
    rem_aligned = ((v - rem_off) // 128) * 128
    if rem_aligned:
        chunks.append((rem_off, rem_aligned))
    tail_off = rem_off + rem_aligned
    tail = v - tail_off

    @functools.partial(
        pl.kernel,
        mesh=mesh,
        out_type=[
            jax.ShapeDtypeStruct((m_rows, b), jnp.float32),
            jax.ShapeDtypeStruct((d * v,), jnp.float32),
        ],
        scratch_types=[
            pltpu.VMEM((b,), jnp.int32),
            pltpu.VMEM((b,), jnp.int32),
            pltpu.VMEM((b,), jnp.float32),
            pltpu.VMEM((b,), jnp.float32),
            pltpu.VMEM((_P1_CHUNK,), jnp.float32),
            pltpu.VMEM((_P1_CHUNK,), jnp.float32),
            pltpu.VMEM((1, max(tail, 1)), jnp.float32),
            pltpu.SemaphoreType.DMA,
            pltpu.SemaphoreType.DMA,
            pltpu.SemaphoreType.DMA,
            pltpu.SemaphoreType.DMA,
            pltpu.SemaphoreType.DMA,
        ],
        compiler_params=pltpu.CompilerParams(needs_layout_passes=False),
    )
    def gather(table_hbm, idx_hbm, out_hbm, scr_hbm, idx0_v, idx1_v, vals0_v,
               vals1_v, chunk0_v, chunk1_v, tail_v, s_in, s_out, s_idx, s_g,
               s_w):
        idx_b = [idx0_v, idx1_v]
        vals_b = [vals0_v, vals1_v]
        chunk_b = [chunk0_v, chunk1_v]
        w = lax.axis_index("s") * _NUM_CORES + lax.axis_index("c")
        row = scr_hbm.at[pl.ds(w * v, v)]
        feat = table_hbm.at[w]

        ci = [None] * l_fields
        for r in range(min(2, l_fields)):
            ci[r] = pltpu.async_copy(idx_hbm.at[r], idx_b[r % 2], s_idx)

        n_ch = len(chunks)
        cin = [None] * n_ch
        cout = [None] * n_ch
        for t in range(min(2, n_ch)):
            off, sz = chunks[t]
            cin[t] = pltpu.async_copy(
                feat.at[pl.ds(off, sz)], chunk_b[t % 2].at[pl.ds(0, sz)], s_in
            )
        for t in range(n_ch):
            off, sz = chunks[t]
            cin[t].wait()
            cout[t] = pltpu.async_copy(
                chunk_b[t % 2].at[pl.ds(0, sz)], row.at[pl.ds(off, sz)], s_out
            )
            if t + 2 < n_ch:
                cout[t].wait()
                off2, sz2 = chunks[t + 2]
                cin[t + 2] = pltpu.async_copy(
                    feat.at[pl.ds(off2, sz2)],
                    chunk_b[t % 2].at[pl.ds(0, sz2)],
                    s_in,
                )
        if tail:
            pltpu.sync_copy(
                table_hbm.at[pl.ds(w, 1), pl.ds(tail_off, tail)], tail_v
            )
            pltpu.sync_copy(tail_v.at[0], row.at[pl.ds(tail_off, tail)])
        for t in range(max(n_ch - 2, 0), n_ch):
            cout[t].wait()

        cg = [None] * l_fields
        cw = [None] * l_fields
        for r in range(l_fields):
            ci[r].wait()
            if r >= 1:
                cg[r - 1].wait()
                cw[r - 1] = pltpu.async_copy(
                    vals_b[(r - 1) % 2], out_hbm.at[(r - 1) * d + w], s_w
                )
                if r + 1 < l_fields:
                    ci[r + 1] = pltpu.async_copy(
                        idx_hbm.at[r + 1], idx_b[(r + 1) % 2], s_idx
                    )
            if r >= 2:
                cw[r - 2].wait()
            cg[r] = pltpu.async_copy(
                row.at[idx_b[r % 2]], vals_b[r % 2], s_g
            )
        cg[l_fields - 1].wait()
        if l_fields >= 2:
            cw[l_fields - 2].wait()
        pltpu.sync_copy(
            vals_b[(l_fields - 1) % 2],
            out_hbm.at[(l_fields - 1) * d + w],
        )

    return gather(table_t, idx_t)[0]


def kernel(inputs, embedding):
    b, l = inputs.shape
    v, d = embedding.shape
    idx_t = inputs.T.astype(jnp.int32)
    table_t = embedding.T
    out_t = _gather_t(table_t, idx_t, v)
    return out_t.T
